# baseline (device time: 24068 ns/iter reference)
import jax
import jax.numpy as jnp
from jax import lax
from jax.experimental import pallas as pl
from jax.experimental.pallas import tpu as pltpu

N_DEV = 4
B, SQ, SKV = 2, 256, 256
H_LOC, DH = 4, 64
D_MODEL = 512
D_CTX = H_LOC * DH


def kernel(x, Wq, K_ext, V_ext, Wo):
    my_pos = lax.axis_index("i")
    wq_p = lax.dynamic_slice(Wq, (0, my_pos * D_CTX), (D_MODEL, D_CTX))
    k2 = K_ext.reshape(B, SKV, D_CTX)
    v2 = V_ext.reshape(B, SKV, D_CTX)

    def body(x_ref, wq_ref, k_ref, v_ref, wo_ref, out_ref,
             comm_ref, send_sems, recv_sems):
        me = lax.axis_index("i")
        left = (me - 1) % N_DEV
        right = (me + 1) % N_DEV

        barrier_sem = pltpu.get_barrier_semaphore()
        for nbr in (left, right):
            pl.semaphore_signal(
                barrier_sem, inc=1,
                device_id=(nbr,), device_id_type=pl.DeviceIdType.MESH,
            )
        pl.semaphore_wait(barrier_sem, 2)

        r = lax.broadcasted_iota(jnp.int32, (SQ, SKV), 0)
        c = lax.broadcasted_iota(jnp.int32, (SQ, SKV), 1)
        qblk, kblk = r // 64, c // 64
        mask = (qblk == kblk) | ((kblk % 4) == (qblk % 4))

        wq = wq_ref[:, :].astype(jnp.bfloat16)
        wo_me = wo_ref[pl.ds(me * D_CTX, D_CTX), :].astype(jnp.bfloat16)
        accs = []
        for b in range(B):
            xb = x_ref[b, :, :].astype(jnp.bfloat16)
            q_all = lax.dot_general(
                xb, wq, (((1,), (0,)), ((), ())),
                preferred_element_type=jnp.float32,
            ).astype(jnp.bfloat16)
            k_all = k_ref[b, :, :].astype(jnp.bfloat16)
            v_all = v_ref[b, :, :].astype(jnp.bfloat16)
            ctx_heads = []
            for h in range(H_LOC):
                qh = q_all[:, h * DH:(h + 1) * DH]
                kh = k_all[:, h * DH:(h + 1) * DH]
                vh = v_all[:, h * DH:(h + 1) * DH]
                s = lax.dot_general(
                    qh, kh, (((1,), (1,)), ((), ())),
                    preferred_element_type=jnp.float32,
                ) * 0.125
                s = jnp.where(mask, s, -1e9)
                m = jnp.max(s, axis=-1, keepdims=True)
                e = jnp.exp(s - m)
                w = (e / jnp.sum(e, axis=-1, keepdims=True)).astype(jnp.bfloat16)
                ctx_heads.append(lax.dot_general(
                    w, vh, (((1,), (0,)), ((), ())),
                    preferred_element_type=jnp.float32,
                ).astype(jnp.bfloat16))
            ctx_b = jnp.concatenate(ctx_heads, axis=1)
            comm_ref[0, b * SQ:(b + 1) * SQ, :] = ctx_b
            accs.append(lax.dot_general(
                ctx_b, wo_me, (((1,), (0,)), ((), ())),
                preferred_element_type=jnp.float32,
            ))

        for hop in range(N_DEV - 1):
            send_slot = hop % 2
            recv_slot = (hop + 1) % 2
            rdma = pltpu.make_async_remote_copy(
                src_ref=comm_ref.at[send_slot],
                dst_ref=comm_ref.at[recv_slot],
                send_sem=send_sems.at[send_slot],
                recv_sem=recv_sems.at[recv_slot],
                device_id=(right,),
                device_id_type=pl.DeviceIdType.MESH,
            )
            rdma.start()
            rdma.wait()
            origin = (me - hop - 1) % N_DEV
            wo_o = wo_ref[pl.ds(origin * D_CTX, D_CTX), :].astype(jnp.bfloat16)
            for b in range(B):
                chunk = comm_ref[recv_slot, b * SQ:(b + 1) * SQ, :]
                accs[b] += lax.dot_general(
                    chunk, wo_o, (((1,), (0,)), ((), ())),
                    preferred_element_type=jnp.float32,
                )

        for b in range(B):
            out_ref[b, :, :] = accs[b]

    return pl.pallas_call(
        body,
        out_shape=jax.ShapeDtypeStruct((B, SQ, D_MODEL), jnp.float32),
        in_specs=[pl.BlockSpec(memory_space=pltpu.VMEM)] * 5,
        out_specs=pl.BlockSpec(memory_space=pltpu.VMEM),
        scratch_shapes=[
            pltpu.VMEM((2, B * SQ, D_CTX), jnp.bfloat16),
            pltpu.SemaphoreType.DMA((2,)),
            pltpu.SemaphoreType.DMA((2,)),
        ],
        compiler_params=pltpu.CompilerParams(collective_id=0),
    )(x, wq_p, k2, v2, Wo)


# device time: 17115 ns/iter; 1.4063x vs baseline; 1.4063x over previous
import jax
import jax.numpy as jnp
from jax import lax
from jax.experimental import pallas as pl
from jax.experimental.pallas import tpu as pltpu

N_DEV = 4
B, SQ, SKV = 2, 256, 256
H_LOC, DH = 4, 64
D_MODEL = 512
D_CTX = H_LOC * DH


def kernel(x, Wq, K_ext, V_ext, Wo):
    my_pos = lax.axis_index("i")
    wq_p = lax.dynamic_slice(Wq, (0, my_pos * D_CTX), (D_MODEL, D_CTX))
    k2 = K_ext.reshape(B, SKV, D_CTX)
    v2 = V_ext.reshape(B, SKV, D_CTX)

    def body(x_ref, wq_ref, k_ref, v_ref, wo_ref, out_ref,
             comm_ref, send_sems, recv_sems):
        me = lax.axis_index("i")

        barrier_sem = pltpu.get_barrier_semaphore()
        for j in range(N_DEV - 1):
            pl.semaphore_signal(
                barrier_sem, inc=1,
                device_id=((me + 1 + j) % N_DEV,),
                device_id_type=pl.DeviceIdType.MESH,
            )

        r = lax.broadcasted_iota(jnp.int32, (SQ, SKV), 0)
        c = lax.broadcasted_iota(jnp.int32, (SQ, SKV), 1)
        qblk, kblk = r // 64, c // 64
        mask = (qblk == kblk) | ((kblk % 4) == (qblk % 4))

        wq = wq_ref[:, :].astype(jnp.bfloat16)
        ctxs = []
        for b in range(B):
            xb = x_ref[b, :, :].astype(jnp.bfloat16)
            q_all = lax.dot_general(
                xb, wq, (((1,), (0,)), ((), ())),
                preferred_element_type=jnp.float32,
            ).astype(jnp.bfloat16)
            k_all = k_ref[b, :, :].astype(jnp.bfloat16)
            v_all = v_ref[b, :, :].astype(jnp.bfloat16)
            ctx_heads = []
            for h in range(H_LOC):
                qh = q_all[:, h * DH:(h + 1) * DH]
                kh = k_all[:, h * DH:(h + 1) * DH]
                vh = v_all[:, h * DH:(h + 1) * DH]
                s = lax.dot_general(
                    qh, kh, (((1,), (1,)), ((), ())),
                    preferred_element_type=jnp.float32,
                ) * 0.125
                s = jnp.where(mask, s, -1e9)
                m = jnp.max(s, axis=-1, keepdims=True)
                e = jnp.exp(s - m)
                w = (e / jnp.sum(e, axis=-1, keepdims=True)).astype(jnp.bfloat16)
                ctx_heads.append(lax.dot_general(
                    w, vh, (((1,), (0,)), ((), ())),
                    preferred_element_type=jnp.float32,
                ).astype(jnp.bfloat16))
            ctxs.append(jnp.concatenate(ctx_heads, axis=1))
            comm_ref[0, b * SQ:(b + 1) * SQ, :] = ctxs[b]

        pl.semaphore_wait(barrier_sem, N_DEV - 1)

        sends = []
        for j in range(N_DEV - 1):
            dst = (me + 1 + j) % N_DEV
            rdma = pltpu.make_async_remote_copy(
                src_ref=comm_ref.at[0],
                dst_ref=comm_ref.at[3 - j],
                send_sem=send_sems.at[j],
                recv_sem=recv_sems.at[3 - j],
                device_id=(dst,),
                device_id_type=pl.DeviceIdType.MESH,
            )
            rdma.start()
            sends.append(rdma)

        wo_me = wo_ref[pl.ds(me * D_CTX, D_CTX), :].astype(jnp.bfloat16)
        accs = [
            lax.dot_general(
                ctxs[b], wo_me, (((1,), (0,)), ((), ())),
                preferred_element_type=jnp.float32,
            )
            for b in range(B)
        ]

        for j in (0, 2, 1):
            origin = (me + 1 + j) % N_DEV
            slot = j + 1
            recv = pltpu.make_async_remote_copy(
                src_ref=comm_ref.at[0],
                dst_ref=comm_ref.at[slot],
                send_sem=send_sems.at[j],
                recv_sem=recv_sems.at[slot],
                device_id=(me,),
                device_id_type=pl.DeviceIdType.MESH,
            )
            recv.wait_recv()
            wo_o = wo_ref[pl.ds(origin * D_CTX, D_CTX), :].astype(jnp.bfloat16)
            for b in range(B):
                chunk = comm_ref[slot, b * SQ:(b + 1) * SQ, :]
                accs[b] += lax.dot_general(
                    chunk, wo_o, (((1,), (0,)), ((), ())),
                    preferred_element_type=jnp.float32,
                )

        for rdma in sends:
            rdma.wait_send()

        for b in range(B):
            out_ref[b, :, :] = accs[b]

    return pl.pallas_call(
        body,
        out_shape=jax.ShapeDtypeStruct((B, SQ, D_MODEL), jnp.float32),
        in_specs=[pl.BlockSpec(memory_space=pltpu.VMEM)] * 5,
        out_specs=pl.BlockSpec(memory_space=pltpu.VMEM),
        scratch_shapes=[
            pltpu.VMEM((N_DEV, B * SQ, D_CTX), jnp.bfloat16),
            pltpu.SemaphoreType.DMA((N_DEV - 1,)),
            pltpu.SemaphoreType.DMA((N_DEV,)),
        ],
        compiler_params=pltpu.CompilerParams(collective_id=0),
    )(x, wq_p, k2, v2, Wo)


# device time: 15904 ns/iter; 1.5133x vs baseline; 1.0761x over previous
import jax
import jax.numpy as jnp
from jax import lax
from jax.experimental import pallas as pl
from jax.experimental.pallas import tpu as pltpu

N_DEV = 4
B, SQ, SKV = 2, 256, 256
H_LOC, DH = 4, 64
D_MODEL = 512
D_CTX = H_LOC * DH


def kernel(x, Wq, K_ext, V_ext, Wo):
    my_pos = lax.axis_index("i")
    wq_p = lax.dynamic_slice(Wq, (0, my_pos * D_CTX), (D_MODEL, D_CTX))
    k2 = K_ext.reshape(B, SKV, D_CTX)
    v2 = V_ext.reshape(B, SKV, D_CTX)

    def body(x_ref, wq_ref, k_ref, v_ref, wo_ref, out_ref,
             comm_ref, send_sems, recv_sems):
        me = lax.axis_index("i")

        barrier_sem = pltpu.get_barrier_semaphore()
        for j in range(N_DEV - 1):
            pl.semaphore_signal(
                barrier_sem, inc=1,
                device_id=((me + 1 + j) % N_DEV,),
                device_id_type=pl.DeviceIdType.MESH,
            )

        r = lax.broadcasted_iota(jnp.int32, (SQ, SKV), 0)
        c = lax.broadcasted_iota(jnp.int32, (SQ, SKV), 1)
        qblk, kblk = r // 64, c // 64
        mask = (qblk == kblk) | ((kblk % 4) == (qblk % 4))

        wq = wq_ref[:, :].astype(jnp.bfloat16)

        def ctx_for_batch(b):
            xb = x_ref[b, :, :].astype(jnp.bfloat16)
            q_all = lax.dot_general(
                xb, wq, (((1,), (0,)), ((), ())),
                preferred_element_type=jnp.float32,
            ).astype(jnp.bfloat16)
            k_all = k_ref[b, :, :].astype(jnp.bfloat16)
            v_all = v_ref[b, :, :].astype(jnp.bfloat16)
            ctx_heads = []
            for h in range(H_LOC):
                qh = q_all[:, h * DH:(h + 1) * DH]
                kh = k_all[:, h * DH:(h + 1) * DH]
                vh = v_all[:, h * DH:(h + 1) * DH]
                s = lax.dot_general(
                    qh, kh, (((1,), (1,)), ((), ())),
                    preferred_element_type=jnp.float32,
                ) * 0.125
                s = jnp.where(mask, s, -1e9)
                m = jnp.max(s, axis=-1, keepdims=True)
                e = jnp.exp(s - m)
                w = (e / jnp.sum(e, axis=-1, keepdims=True)).astype(jnp.bfloat16)
                ctx_heads.append(lax.dot_general(
                    w, vh, (((1,), (0,)), ((), ())),
                    preferred_element_type=jnp.float32,
                ).astype(jnp.bfloat16))
            return jnp.concatenate(ctx_heads, axis=1)

        def start_half_sends(half):
            out = []
            for j in (1, 0, 2):
                dst = (me + 1 + j) % N_DEV
                rdma = pltpu.make_async_remote_copy(
                    src_ref=comm_ref.at[0, pl.ds(half * SQ, SQ)],
                    dst_ref=comm_ref.at[3 - j, pl.ds(half * SQ, SQ)],
                    send_sem=send_sems.at[j, half],
                    recv_sem=recv_sems.at[3 - j, half],
                    device_id=(dst,),
                    device_id_type=pl.DeviceIdType.MESH,
                )
                rdma.start()
                out.append(rdma)
            return out

        ctx0 = ctx_for_batch(0)
        comm_ref[0, 0:SQ, :] = ctx0
        pl.semaphore_wait(barrier_sem, N_DEV - 1)
        sends = start_half_sends(0)

        ctx1 = ctx_for_batch(1)
        comm_ref[0, SQ:2 * SQ, :] = ctx1
        sends += start_half_sends(1)

        ctx_full = jnp.concatenate([ctx0, ctx1], axis=0)
        wo_me = wo_ref[pl.ds(me * D_CTX, D_CTX), :].astype(jnp.bfloat16)
        acc = lax.dot_general(
            ctx_full, wo_me, (((1,), (0,)), ((), ())),
            preferred_element_type=jnp.float32,
        )

        for j in (0, 2, 1):
            origin = (me + 1 + j) % N_DEV
            slot = j + 1
            for half in range(2):
                recv = pltpu.make_async_remote_copy(
                    src_ref=comm_ref.at[0, pl.ds(half * SQ, SQ)],
                    dst_ref=comm_ref.at[slot, pl.ds(half * SQ, SQ)],
                    send_sem=send_sems.at[j, half],
                    recv_sem=recv_sems.at[slot, half],
                    device_id=(me,),
                    device_id_type=pl.DeviceIdType.MESH,
                )
                recv.wait_recv()
            wo_o = wo_ref[pl.ds(origin * D_CTX, D_CTX), :].astype(jnp.bfloat16)
            chunk = comm_ref[slot, :, :]
            acc += lax.dot_general(
                chunk, wo_o, (((1,), (0,)), ((), ())),
                preferred_element_type=jnp.float32,
            )

        for rdma in sends:
            rdma.wait_send()

        for b in range(B):
            out_ref[b, :, :] = acc[b * SQ:(b + 1) * SQ, :]

    return pl.pallas_call(
        body,
        out_shape=jax.ShapeDtypeStruct((B, SQ, D_MODEL), jnp.float32),
        in_specs=[pl.BlockSpec(memory_space=pltpu.VMEM)] * 5,
        out_specs=pl.BlockSpec(memory_space=pltpu.VMEM),
        scratch_shapes=[
            pltpu.VMEM((N_DEV, B * SQ, D_CTX), jnp.bfloat16),
            pltpu.SemaphoreType.DMA((N_DEV - 1, 2)),
            pltpu.SemaphoreType.DMA((N_DEV, 2)),
        ],
        compiler_params=pltpu.CompilerParams(collective_id=0),
    )(x, wq_p, k2, v2, Wo)
